# Initial kernel scaffold; baseline (speedup 1.0000x reference)
import jax
import jax.numpy as jnp
from jax.experimental import pallas as pl


def kernel(src_node_type, dst_node_type, embedding):
    raise NotImplementedError("write your pallas kernel here")


# speedup vs baseline: 2.7957x; 2.7957x over previous
import functools

import jax
import jax.numpy as jnp
from jax import lax
from jax.experimental import pallas as pl
from jax.experimental.pallas import tpu as pltpu
from jax.experimental.pallas import tpu_sc as plsc

E = 320000
DIM = 128
NUM_CORES = 2
NUM_SUBCORES = 16
NW = NUM_CORES * NUM_SUBCORES
B_PER_W = E // NW
CHUNK = 400
NCH = B_PER_W // CHUNK
G = 80
NG = CHUNK // G
LANES = 16


def _body(src_hbm, dst_hbm, table_hbm, out_hbm, srcv, dstv, idxv, rows, sem):
    wid = lax.axis_index("s") * NUM_CORES + lax.axis_index("c")
    base = wid * B_PER_W

    def chunk_body(i, carry):
        row0 = base + i * CHUNK
        pltpu.sync_copy(src_hbm.at[pl.ds(row0, CHUNK)], srcv)
        pltpu.sync_copy(dst_hbm.at[pl.ds(row0, CHUNK)], dstv)

        def compute(j, c):
            x = srcv[pl.ds(j * LANES, LANES)]
            y = dstv[pl.ds(j * LANES, LANES)]
            d = jnp.abs(x - y) - 1
            idxv[pl.ds(j * LANES, LANES)] = x * y + ((d * d) >> 2)
            return c

        lax.fori_loop(0, CHUNK // LANES, compute, 0)

        for g in range(NG):
            pltpu.async_copy(
                table_hbm.at[idxv.at[pl.ds(g * G, G)]],
                rows.at[pl.ds(g * G, G)],
                sem,
            ).wait()

        pltpu.sync_copy(rows, out_hbm.at[pl.ds(row0, CHUNK)])
        return carry

    lax.fori_loop(0, NCH, chunk_body, 0)


@functools.partial(jax.jit, static_argnums=())
def _run(src, dst, table):
    mesh = plsc.VectorSubcoreMesh(core_axis_name="c", subcore_axis_name="s")
    f = functools.partial(
        pl.kernel,
        mesh=mesh,
        out_type=jax.ShapeDtypeStruct((E, DIM), jnp.float32),
        scratch_types=[
            pltpu.VMEM((CHUNK,), jnp.int32),
            pltpu.VMEM((CHUNK,), jnp.int32),
            pltpu.VMEM((CHUNK,), jnp.int32),
            pltpu.VMEM((CHUNK, DIM), jnp.float32),
            pltpu.SemaphoreType.DMA,
        ],
    )(_body)
    return f(src, dst, table)


def kernel(src_node_type, dst_node_type, embedding):
    src = src_node_type.astype(jnp.int32)
    dst = dst_node_type.astype(jnp.int32)
    table = embedding.astype(jnp.float32)
    return _run(src, dst, table)


# speedup vs baseline: 3.6973x; 1.3225x over previous
import functools

import jax
import jax.numpy as jnp
from jax import lax
from jax.experimental import pallas as pl
from jax.experimental.pallas import tpu as pltpu
from jax.experimental.pallas import tpu_sc as plsc

E = 320000
DIM = 128
NUM_CORES = 2
NUM_SUBCORES = 16
NW = NUM_CORES * NUM_SUBCORES
B_PER_W = E // NW
CHUNK = 400
NCH = B_PER_W // CHUNK
G = 80
NG = CHUNK // G
LANES = 16


def _body(src_hbm, dst_hbm, table_hbm, out_hbm,
          src0, src1, dst0, dst1, idx0, idx1, rows0, rows1,
          isem0, isem1, gsem0, gsem1, osem0, osem1):
    src_b = (src0, src1)
    dst_b = (dst0, dst1)
    idx_b = (idx0, idx1)
    rows_b = (rows0, rows1)
    isem = (isem0, isem1)
    gsem = (gsem0, gsem1)
    osem = (osem0, osem1)

    wid = lax.axis_index("s") * NUM_CORES + lax.axis_index("c")
    base = wid * B_PER_W

    in_desc = [None] * NCH
    g_desc = [[None] * NG for _ in range(NCH)]
    o_desc = [None] * NCH

    def fire_inputs(i):
        b = i % 2
        row0 = base + i * CHUNK
        in_desc[i] = (
            pltpu.async_copy(src_hbm.at[pl.ds(row0, CHUNK)], src_b[b], isem[b]),
            pltpu.async_copy(dst_hbm.at[pl.ds(row0, CHUNK)], dst_b[b], isem[b]),
        )

    def compute(i):
        b = i % 2
        in_desc[i][0].wait()
        in_desc[i][1].wait()

        def f(j, c):
            x = src_b[b][pl.ds(j * LANES, LANES)]
            y = dst_b[b][pl.ds(j * LANES, LANES)]
            d = jnp.abs(x - y) - 1
            idx_b[b][pl.ds(j * LANES, LANES)] = x * y + ((d * d) >> 2)
            return c

        lax.fori_loop(0, CHUNK // LANES, f, 0)

    def fire_gathers(i):
        b = i % 2
        for g in range(NG):
            g_desc[i][g] = pltpu.async_copy(
                table_hbm.at[idx_b[b].at[pl.ds(g * G, G)]],
                rows_b[b].at[pl.ds(g * G, G)],
                gsem[b],
            )

    def fire_write(i):
        b = i % 2
        for gd in g_desc[i]:
            gd.wait()
        o_desc[i] = pltpu.async_copy(
            rows_b[b], out_hbm.at[pl.ds(base + i * CHUNK, CHUNK)], osem[b]
        )

    fire_inputs(0)
    fire_inputs(1)
    compute(0)
    fire_gathers(0)
    for i in range(NCH):
        if i + 2 < NCH:
            fire_inputs(i + 2)
        if i + 1 < NCH:
            compute(i + 1)
            if i - 1 >= 0:
                o_desc[i - 1].wait()
            fire_gathers(i + 1)
        fire_write(i)
    o_desc[NCH - 2].wait()
    o_desc[NCH - 1].wait()


@jax.jit
def _run(src, dst, table):
    mesh = plsc.VectorSubcoreMesh(core_axis_name="c", subcore_axis_name="s")
    f = functools.partial(
        pl.kernel,
        mesh=mesh,
        out_type=jax.ShapeDtypeStruct((E, DIM), jnp.float32),
        scratch_types=[
            pltpu.VMEM((CHUNK,), jnp.int32),
            pltpu.VMEM((CHUNK,), jnp.int32),
            pltpu.VMEM((CHUNK,), jnp.int32),
            pltpu.VMEM((CHUNK,), jnp.int32),
            pltpu.VMEM((CHUNK,), jnp.int32),
            pltpu.VMEM((CHUNK,), jnp.int32),
            pltpu.VMEM((CHUNK, DIM), jnp.float32),
            pltpu.VMEM((CHUNK, DIM), jnp.float32),
            pltpu.SemaphoreType.DMA,
            pltpu.SemaphoreType.DMA,
            pltpu.SemaphoreType.DMA,
            pltpu.SemaphoreType.DMA,
            pltpu.SemaphoreType.DMA,
            pltpu.SemaphoreType.DMA,
        ],
    )(_body)
    return f(src, dst, table)


def kernel(src_node_type, dst_node_type, embedding):
    src = src_node_type.astype(jnp.int32)
    dst = dst_node_type.astype(jnp.int32)
    table = embedding.astype(jnp.float32)
    return _run(src, dst, table)


# speedup vs baseline: 8.1757x; 2.2113x over previous
import functools

import jax
import jax.numpy as jnp
from jax import lax
from jax.experimental import pallas as pl
from jax.experimental.pallas import tpu as pltpu
from jax.experimental.pallas import tpu_sc as plsc

E = 320000
DIM = 128
NUM_CORES = 2
NUM_SUBCORES = 16
NW = NUM_CORES * NUM_SUBCORES
B_PER_W = E // NW
CHUNK = 400
NCH = B_PER_W // CHUNK
G = 80
NG = CHUNK // G
LANES = 16


def _body(src_hbm, dst_hbm, table_hbm, out_hbm,
          src0, src1, dst0, dst1, idx0, idx1, rows0, rows1, table_sp,
          isem0, isem1, gsem0, gsem1, osem0, osem1):
    src_b = (src0, src1)
    dst_b = (dst0, dst1)
    idx_b = (idx0, idx1)
    rows_b = (rows0, rows1)
    isem = (isem0, isem1)
    gsem = (gsem0, gsem1)
    osem = (osem0, osem1)

    wid = lax.axis_index("s") * NUM_CORES + lax.axis_index("c")
    base = wid * B_PER_W

    @pl.when(lax.axis_index("s") == 0)
    def _():
        pltpu.sync_copy(table_hbm, table_sp)

    plsc.subcore_barrier()

    in_desc = [None] * NCH
    g_desc = [[None] * NG for _ in range(NCH)]
    o_desc = [None] * NCH

    def fire_inputs(i):
        b = i % 2
        row0 = base + i * CHUNK
        in_desc[i] = (
            pltpu.async_copy(src_hbm.at[pl.ds(row0, CHUNK)], src_b[b], isem[b]),
            pltpu.async_copy(dst_hbm.at[pl.ds(row0, CHUNK)], dst_b[b], isem[b]),
        )

    def compute(i):
        b = i % 2
        in_desc[i][0].wait()
        in_desc[i][1].wait()

        def f(j, c):
            x = src_b[b][pl.ds(j * LANES, LANES)]
            y = dst_b[b][pl.ds(j * LANES, LANES)]
            d = jnp.abs(x - y) - 1
            idx_b[b][pl.ds(j * LANES, LANES)] = x * y + ((d * d) >> 2)
            return c

        lax.fori_loop(0, CHUNK // LANES, f, 0)

    def fire_gathers(i):
        b = i % 2
        for g in range(NG):
            g_desc[i][g] = pltpu.async_copy(
                table_sp.at[idx_b[b].at[pl.ds(g * G, G)]],
                rows_b[b].at[pl.ds(g * G, G)],
                gsem[b],
            )

    def fire_write(i):
        b = i % 2
        for gd in g_desc[i]:
            gd.wait()
        o_desc[i] = pltpu.async_copy(
            rows_b[b], out_hbm.at[pl.ds(base + i * CHUNK, CHUNK)], osem[b]
        )

    fire_inputs(0)
    fire_inputs(1)
    compute(0)
    fire_gathers(0)
    for i in range(NCH):
        if i + 2 < NCH:
            fire_inputs(i + 2)
        if i + 1 < NCH:
            compute(i + 1)
            if i - 1 >= 0:
                o_desc[i - 1].wait()
            fire_gathers(i + 1)
        fire_write(i)
    o_desc[NCH - 2].wait()
    o_desc[NCH - 1].wait()


@jax.jit
def _run(src, dst, table):
    mesh = plsc.VectorSubcoreMesh(core_axis_name="c", subcore_axis_name="s")
    f = functools.partial(
        pl.kernel,
        mesh=mesh,
        out_type=jax.ShapeDtypeStruct((E, DIM), jnp.float32),
        scratch_types=[
            pltpu.VMEM((CHUNK,), jnp.int32),
            pltpu.VMEM((CHUNK,), jnp.int32),
            pltpu.VMEM((CHUNK,), jnp.int32),
            pltpu.VMEM((CHUNK,), jnp.int32),
            pltpu.VMEM((CHUNK,), jnp.int32),
            pltpu.VMEM((CHUNK,), jnp.int32),
            pltpu.VMEM((CHUNK, DIM), jnp.float32),
            pltpu.VMEM((CHUNK, DIM), jnp.float32),
            pltpu.VMEM_SHARED((3000, DIM), jnp.float32),
            pltpu.SemaphoreType.DMA,
            pltpu.SemaphoreType.DMA,
            pltpu.SemaphoreType.DMA,
            pltpu.SemaphoreType.DMA,
            pltpu.SemaphoreType.DMA,
            pltpu.SemaphoreType.DMA,
        ],
    )(_body)
    return f(src, dst, table)


def kernel(src_node_type, dst_node_type, embedding):
    src = src_node_type.astype(jnp.int32)
    dst = dst_node_type.astype(jnp.int32)
    table = embedding.astype(jnp.float32)
    return _run(src, dst, table)


# speedup vs baseline: 8.4609x; 1.0349x over previous
import functools

import jax
import jax.numpy as jnp
from jax import lax
from jax.experimental import pallas as pl
from jax.experimental.pallas import tpu as pltpu
from jax.experimental.pallas import tpu_sc as plsc

E = 320000
DIM = 128
TBL = 3000
NUM_CORES = 2
NUM_SUBCORES = 16
NW = NUM_CORES * NUM_SUBCORES
B_PER_W = E // NW
CHUNK = 400
NCH = B_PER_W // CHUNK
LANES = 16


def _body(src_hbm, dst_hbm, table_hbm, out_hbm,
          src0, src1, dst0, dst1, idxv, rows0, rows1, table_sp,
          isem0, isem1, gsem0, gsem1, osem0, osem1):
    src_b = (src0, src1)
    dst_b = (dst0, dst1)
    rows_b = (rows0, rows1)
    isem = (isem0, isem1)
    gsem = (gsem0, gsem1)
    osem = (osem0, osem1)

    wid = lax.axis_index("s") * NUM_CORES + lax.axis_index("c")
    base = wid * B_PER_W

    def fire_in(i, b):
        row0 = base + i * CHUNK
        pltpu.async_copy(src_hbm.at[pl.ds(row0, CHUNK)], src_b[b], isem[b])
        pltpu.async_copy(dst_hbm.at[pl.ds(row0, CHUNK)], dst_b[b], isem[b])

    def wait_in(i, b):
        row0 = base + i * CHUNK
        pltpu.make_async_copy(src_hbm.at[pl.ds(row0, CHUNK)], src_b[b], isem[b]).wait()
        pltpu.make_async_copy(dst_hbm.at[pl.ds(row0, CHUNK)], dst_b[b], isem[b]).wait()

    def compute(b):
        def f(j, c):
            x = src_b[b][pl.ds(j * LANES, LANES)]
            y = dst_b[b][pl.ds(j * LANES, LANES)]
            d = jnp.abs(x - y) - 1
            idxv[pl.ds(j * LANES, LANES)] = x * y + ((d * d) >> 2)
            return c

        lax.fori_loop(0, CHUNK // LANES, f, 0)

    def gather(b):
        pltpu.async_copy(table_sp.at[idxv], rows_b[b], gsem[b]).wait()

    def fire_write(i, b):
        pltpu.async_copy(
            rows_b[b], out_hbm.at[pl.ds(base + i * CHUNK, CHUNK)], osem[b]
        )

    def wait_write(i, b):
        pltpu.make_async_copy(
            rows_b[b], out_hbm.at[pl.ds(base + i * CHUNK, CHUNK)], osem[b]
        ).wait()

    fire_in(0, 0)
    fire_in(1, 1)

    @pl.when(lax.axis_index("s") == 0)
    def _():
        pltpu.sync_copy(table_hbm, table_sp)

    plsc.subcore_barrier()

    wait_in(0, 0)
    compute(0)
    fire_in(2, 0)
    gather(0)
    fire_write(0, 0)

    wait_in(1, 1)
    compute(1)
    fire_in(3, 1)
    gather(1)
    fire_write(1, 1)

    def pair(it, c):
        a = 2 + 2 * it
        wait_in(a, 0)
        compute(0)
        fire_in(a + 2, 0)
        wait_write(a - 2, 0)
        gather(0)
        fire_write(a, 0)

        wait_in(a + 1, 1)
        compute(1)
        fire_in(a + 3, 1)
        wait_write(a - 1, 1)
        gather(1)
        fire_write(a + 1, 1)
        return c

    lax.fori_loop(0, (NCH - 5) // 2, pair, 0)

    wait_in(NCH - 3, 0)
    compute(0)
    fire_in(NCH - 1, 0)
    wait_write(NCH - 5, 0)
    gather(0)
    fire_write(NCH - 3, 0)

    wait_in(NCH - 2, 1)
    compute(1)
    wait_write(NCH - 4, 1)
    gather(1)
    fire_write(NCH - 2, 1)

    wait_in(NCH - 1, 0)
    compute(0)
    wait_write(NCH - 3, 0)
    gather(0)
    fire_write(NCH - 1, 0)

    wait_write(NCH - 2, 1)
    wait_write(NCH - 1, 0)


@jax.jit
def _run(src, dst, table):
    mesh = plsc.VectorSubcoreMesh(core_axis_name="c", subcore_axis_name="s")
    f = functools.partial(
        pl.kernel,
        mesh=mesh,
        out_type=jax.ShapeDtypeStruct((E, DIM), jnp.float32),
        scratch_types=[
            pltpu.VMEM((CHUNK,), jnp.int32),
            pltpu.VMEM((CHUNK,), jnp.int32),
            pltpu.VMEM((CHUNK,), jnp.int32),
            pltpu.VMEM((CHUNK,), jnp.int32),
            pltpu.VMEM((CHUNK,), jnp.int32),
            pltpu.VMEM((CHUNK, DIM), jnp.float32),
            pltpu.VMEM((CHUNK, DIM), jnp.float32),
            pltpu.VMEM_SHARED((TBL, DIM), jnp.float32),
            pltpu.SemaphoreType.DMA,
            pltpu.SemaphoreType.DMA,
            pltpu.SemaphoreType.DMA,
            pltpu.SemaphoreType.DMA,
            pltpu.SemaphoreType.DMA,
            pltpu.SemaphoreType.DMA,
        ],
    )(_body)
    return f(src, dst, table)


def kernel(src_node_type, dst_node_type, embedding):
    src = src_node_type.astype(jnp.int32)
    dst = dst_node_type.astype(jnp.int32)
    table = embedding.astype(jnp.float32)
    return _run(src, dst, table)
